# baseline (speedup 1.0000x reference)
import functools

import jax
import jax.numpy as jnp
from jax import lax
from jax.experimental import pallas as pl
from jax.experimental.pallas import tpu as pltpu
from jax.experimental.pallas import tpu_sc as plsc

C = 128
TEXT_DIM = 1536
N_USERS = 4096
N_ITEMS = 4096
N_NUM = 4
N_CAT = 4
N_TXT = 1
VOCAB = 101
TOK = N_NUM + N_CAT + N_TXT

NC, NS = 2, 16
NW = NC * NS
TOTAL_CAT = 2 * N_CAT * N_USERS
ROWS_PER_W = TOTAL_CAT // NW
GCHUNK = 128
NCHUNK = ROWS_PER_W // GCHUNK
HALF = NCHUNK // 2


def _sc_gather(all_tab, idx3):
    mesh = plsc.VectorSubcoreMesh(core_axis_name="c", subcore_axis_name="s")

    @functools.partial(
        pl.kernel,
        mesh=mesh,
        out_type=jax.ShapeDtypeStruct((TOTAL_CAT, C), jnp.float32),
        scratch_types=[
            pltpu.VMEM((NCHUNK, GCHUNK), jnp.int32),
            pltpu.VMEM((HALF * GCHUNK, C), jnp.float32),
            pltpu.SemaphoreType.DMA,
        ],
    )
    def k(tab_hbm, idx_hbm, out_hbm, idx_v, rows_v, sem):
        wid = lax.axis_index("s") * NC + lax.axis_index("c")
        base = wid * ROWS_PER_W
        pltpu.sync_copy(idx_hbm.at[wid], idx_v)
        for h in range(NCHUNK // HALF):
            cps = [
                pltpu.async_copy(
                    tab_hbm.at[idx_v.at[h * HALF + j]],
                    rows_v.at[pl.ds(j * GCHUNK, GCHUNK)],
                    sem,
                )
                for j in range(HALF)
            ]
            for cp in cps:
                cp.wait()
            pltpu.sync_copy(
                rows_v, out_hbm.at[pl.ds(base + h * HALF * GCHUNK, HALF * GCHUNK)]
            )

    return k(all_tab, idx3)


R = 512
NRT = N_USERS // R


def _tc_body(nfj_ref, ut_ref, it_ref, tw_ref, nw_ref, nb_ref, av_ref, cat_ref,
             o_ref):
    b = pl.program_id(0)

    bmask = lax.broadcasted_iota(jnp.int32, (24, 1), 0) == b
    addv = jnp.sum(av_ref[...] * bmask, axis=0, keepdims=True)

    def num_branch():
        side = b >= 9
        widx = jnp.where(side, b - 5, b)
        cmask = lax.broadcasted_iota(jnp.int32, (1, 8), 1) == widx
        z = jnp.sum(nfj_ref[...] * cmask, axis=1, keepdims=True)
        wmask = lax.broadcasted_iota(jnp.int32, (8, 1), 0) == widx
        wrow = jnp.sum(nw_ref[...] * wmask, axis=0, keepdims=True)
        brow = jnp.sum(nb_ref[...] * wmask, axis=0, keepdims=True)
        zz = z * wrow + brow
        o_ref[0] = zz / (1.0 + jnp.exp(-zz)) + addv

    def cat_branch():
        o_ref[0] = cat_ref[0] + addv

    def text_branch():
        feat = jnp.where(b == 8, ut_ref[...], it_ref[...])
        o_ref[0] = jnp.dot(feat, tw_ref[0],
                           preferred_element_type=jnp.float32) + addv

    btype = jnp.where(
        b < 4, 0, jnp.where(b < 8, 1, jnp.where(
            b == 8, 2, jnp.where(b < 13, 0, jnp.where(b < 17, 1, 2)))))
    lax.switch(btype, (num_branch, cat_branch, text_branch))


def _tc_call(nf_joint, ut2, it2, textW, numW_all, numb_all, addvec, cat3):
    def nfj_idx(b, r):
        return (jnp.where(b < 4, r, jnp.where(b < 9, 7, jnp.where(b < 13, r, 7))), 0)

    def ut_idx(b, r):
        return (jnp.where(b < 8, 0, jnp.where(b == 8, r, 7)), 0)

    def it_idx(b, r):
        return (jnp.where(b < 17, 0, r), 0)

    def tw_idx(b, r):
        return (jnp.where(b < 17, 0, 1), 0, 0)

    def cat_idx(b, r):
        is_uc = jnp.logical_and(b >= 4, b < 8)
        is_ic = jnp.logical_and(b >= 13, b < 17)
        t = jnp.where(is_uc, b - 4, jnp.where(is_ic, b - 9,
                                              jnp.where(b < 4, 0, jnp.where(b < 13, 3, 7))))
        rr = jnp.where(jnp.logical_or(is_uc, is_ic), r,
                       jnp.where(b < 4, 0, 7))
        return (t, rr, 0)

    return pl.pallas_call(
        _tc_body,
        grid=(2 * TOK, NRT),
        in_specs=[
            pl.BlockSpec((R, 8), nfj_idx),
            pl.BlockSpec((R, TEXT_DIM), ut_idx),
            pl.BlockSpec((R, TEXT_DIM), it_idx),
            pl.BlockSpec((1, TEXT_DIM, C), tw_idx),
            pl.BlockSpec((8, C), lambda b, r: (0, 0)),
            pl.BlockSpec((8, C), lambda b, r: (0, 0)),
            pl.BlockSpec((24, C), lambda b, r: (0, 0)),
            pl.BlockSpec((1, R, C), cat_idx),
        ],
        out_specs=pl.BlockSpec((1, R, C), lambda b, r: (b, r, 0)),
        out_shape=jax.ShapeDtypeStruct((2 * TOK, N_USERS, C), jnp.float32),
    )(nf_joint, ut2, it2, textW, numW_all, numb_all, addvec, cat3)


def kernel(users_num, users_cat, users_text, items_num, items_cat, items_text,
           table_emb, u_num_W, u_num_b, u_num_col, u_cat_tab, u_cat_col,
           u_text_W, u_text_b, u_text_col, i_num_W, i_num_b, i_num_col,
           i_cat_tab, i_cat_col, i_text_W, i_text_b, i_text_col):
    ut2 = users_text.reshape(N_USERS, TEXT_DIM)
    it2 = items_text.reshape(N_ITEMS, TEXT_DIM)
    textW = jnp.stack([u_text_W[0], i_text_W[0]])
    numW_all = jnp.concatenate([u_num_W[:, 0, :], i_num_W[:, 0, :]])
    numb_all = jnp.concatenate([u_num_b, i_num_b])
    nf_joint = jnp.concatenate([users_num, items_num], axis=1)
    te_u, te_i = table_emb[0], table_emb[1]
    addvec = jnp.concatenate([
        u_num_col + te_u,
        u_cat_col + te_u,
        u_text_col + u_text_b + te_u,
        i_num_col + te_i,
        i_cat_col + te_i,
        i_text_col + i_text_b + te_i,
        jnp.zeros((24 - 2 * TOK, C), jnp.float32),
    ])

    all_tab = jnp.concatenate([u_cat_tab, i_cat_tab]).reshape(2 * N_CAT * VOCAB, C)
    idx = jnp.concatenate([users_cat.T, items_cat.T]).astype(jnp.int32)
    idx = idx + (jnp.arange(2 * N_CAT, dtype=jnp.int32) * VOCAB)[:, None]
    idx3 = idx.reshape(NW, NCHUNK, GCHUNK)

    cat_rows = _sc_gather(all_tab, idx3)

    x3 = _tc_call(nf_joint, ut2, it2, textW, numW_all, numb_all, addvec,
                  cat_rows.reshape(2 * N_CAT, N_USERS, C))
    x = x3.reshape(2 * TOK * N_USERS, C)

    node_idxs = jnp.concatenate([
        jnp.tile(jnp.arange(N_USERS), TOK),
        jnp.tile(jnp.arange(N_USERS, N_USERS + N_ITEMS), TOK),
    ])
    table_idxs = jnp.concatenate([
        jnp.zeros(N_USERS * TOK, dtype=jnp.int32),
        jnp.ones(N_ITEMS * TOK, dtype=jnp.int32),
    ])
    col_parts = ([jnp.full((N_USERS,), c, dtype=jnp.int32) for c in range(TOK)]
                 + [jnp.full((N_ITEMS,), TOK + c, dtype=jnp.int32) for c in range(TOK)])
    col_idxs = jnp.concatenate(col_parts)
    return (x, node_idxs, col_idxs, table_idxs, N_USERS + N_ITEMS)


# speedup vs baseline: 1.6187x; 1.6187x over previous
import functools

import jax
import jax.numpy as jnp
from jax import lax
from jax.experimental import pallas as pl
from jax.experimental.pallas import tpu as pltpu
from jax.experimental.pallas import tpu_sc as plsc

C = 128
TEXT_DIM = 1536
N_USERS = 4096
N_ITEMS = 4096
N_NUM = 4
N_CAT = 4
N_TXT = 1
VOCAB = 101
TOK = N_NUM + N_CAT + N_TXT

NC, NS = 2, 16
NW = NC * NS
TOTAL_CAT = 2 * N_CAT * N_USERS
ROWS_PER_W = TOTAL_CAT // NW
GCHUNK = 128
NCHUNK = ROWS_PER_W // GCHUNK
HALF = NCHUNK // 2


def _sc_gather(all_tab, idx3):
    mesh = plsc.VectorSubcoreMesh(core_axis_name="c", subcore_axis_name="s")

    @functools.partial(
        pl.kernel,
        mesh=mesh,
        out_type=jax.ShapeDtypeStruct((TOTAL_CAT, C), jnp.float32),
        scratch_types=[
            pltpu.VMEM((NCHUNK, GCHUNK), jnp.int32),
            pltpu.VMEM((HALF * GCHUNK, C), jnp.float32),
            pltpu.SemaphoreType.DMA,
        ],
    )
    def k(tab_hbm, idx_hbm, out_hbm, idx_v, rows_v, sem):
        wid = lax.axis_index("s") * NC + lax.axis_index("c")
        base = wid * ROWS_PER_W
        pltpu.sync_copy(idx_hbm.at[wid], idx_v)
        for h in range(NCHUNK // HALF):
            cps = [
                pltpu.async_copy(
                    tab_hbm.at[idx_v.at[h * HALF + j]],
                    rows_v.at[pl.ds(j * GCHUNK, GCHUNK)],
                    sem,
                )
                for j in range(HALF)
            ]
            for cp in cps:
                cp.wait()
            pltpu.sync_copy(
                rows_v, out_hbm.at[pl.ds(base + h * HALF * GCHUNK, HALF * GCHUNK)]
            )

    return k(all_tab, idx3)


R = 512
NRT = N_USERS // R


def _tc_body(nfj_ref, ut_ref, it_ref, tw_ref, nw_ref, nb_ref, av_ref, cat_ref,
             o_ref):
    s = pl.program_id(0)

    av = av_ref[...]
    riota = lax.broadcasted_iota(jnp.int32, (24, 1), 0)

    def avrow(blk):
        return jnp.sum(av * (riota == blk), axis=0, keepdims=True)

    w8iota = lax.broadcasted_iota(jnp.int32, (8, 1), 0)
    c8iota = lax.broadcasted_iota(jnp.int32, (1, 8), 1)
    nfj = nfj_ref[...]
    nw = nw_ref[...]
    nb = nb_ref[...]

    for i in range(N_NUM):
        widx = s * 4 + i
        z = jnp.sum(nfj * (c8iota == widx), axis=1, keepdims=True)
        wrow = jnp.sum(nw * (w8iota == widx), axis=0, keepdims=True)
        brow = jnp.sum(nb * (w8iota == widx), axis=0, keepdims=True)
        zz = z * wrow + brow
        o_ref[0, i] = zz / (1.0 + jnp.exp(-zz)) + avrow(s * TOK + i)

    for i in range(N_CAT):
        o_ref[0, N_NUM + i] = cat_ref[i] + avrow(s * TOK + N_NUM + i)

    feat = jnp.where(s == 0, ut_ref[...], it_ref[...])
    o_ref[0, 2 * N_NUM] = (
        jnp.dot(feat, tw_ref[0], preferred_element_type=jnp.float32)
        + avrow(s * TOK + 2 * N_NUM))


def _tc_call(nf_joint, ut2, it2, textW, numW_all, numb_all, addvec, cat3):
    return pl.pallas_call(
        _tc_body,
        grid=(2, NRT),
        in_specs=[
            pl.BlockSpec((R, 8), lambda s, r: (r, 0)),
            pl.BlockSpec((R, TEXT_DIM), lambda s, r: (jnp.where(s == 0, r, NRT - 1), 0)),
            pl.BlockSpec((R, TEXT_DIM), lambda s, r: (jnp.where(s == 0, 0, r), 0)),
            pl.BlockSpec((1, TEXT_DIM, C), lambda s, r: (s, 0, 0)),
            pl.BlockSpec((8, C), lambda s, r: (0, 0)),
            pl.BlockSpec((8, C), lambda s, r: (0, 0)),
            pl.BlockSpec((24, C), lambda s, r: (0, 0)),
            pl.BlockSpec((N_CAT, R, C), lambda s, r: (s, r, 0)),
        ],
        out_specs=pl.BlockSpec((1, TOK, R, C), lambda s, r: (s, 0, r, 0)),
        out_shape=jax.ShapeDtypeStruct((2, TOK, N_USERS, C), jnp.float32),
    )(nf_joint, ut2, it2, textW, numW_all, numb_all, addvec, cat3)


def kernel(users_num, users_cat, users_text, items_num, items_cat, items_text,
           table_emb, u_num_W, u_num_b, u_num_col, u_cat_tab, u_cat_col,
           u_text_W, u_text_b, u_text_col, i_num_W, i_num_b, i_num_col,
           i_cat_tab, i_cat_col, i_text_W, i_text_b, i_text_col):
    ut2 = users_text.reshape(N_USERS, TEXT_DIM)
    it2 = items_text.reshape(N_ITEMS, TEXT_DIM)
    textW = jnp.stack([u_text_W[0], i_text_W[0]])
    numW_all = jnp.concatenate([u_num_W[:, 0, :], i_num_W[:, 0, :]])
    numb_all = jnp.concatenate([u_num_b, i_num_b])
    nf_joint = jnp.concatenate([users_num, items_num], axis=1)
    te_u, te_i = table_emb[0], table_emb[1]
    addvec = jnp.concatenate([
        u_num_col + te_u,
        u_cat_col + te_u,
        u_text_col + u_text_b + te_u,
        i_num_col + te_i,
        i_cat_col + te_i,
        i_text_col + i_text_b + te_i,
        jnp.zeros((24 - 2 * TOK, C), jnp.float32),
    ])

    all_tab = jnp.concatenate([u_cat_tab, i_cat_tab]).reshape(2 * N_CAT * VOCAB, C)
    idx = jnp.concatenate([users_cat.T, items_cat.T]).astype(jnp.int32)
    idx = idx + (jnp.arange(2 * N_CAT, dtype=jnp.int32) * VOCAB)[:, None]
    idx3 = idx.reshape(NW, NCHUNK, GCHUNK)

    cat_rows = _sc_gather(all_tab, idx3)

    x3 = _tc_call(nf_joint, ut2, it2, textW, numW_all, numb_all, addvec,
                  cat_rows.reshape(2 * N_CAT, N_USERS, C))
    x = x3.reshape(2 * TOK * N_USERS, C)

    node_idxs = jnp.concatenate([
        jnp.tile(jnp.arange(N_USERS), TOK),
        jnp.tile(jnp.arange(N_USERS, N_USERS + N_ITEMS), TOK),
    ])
    table_idxs = jnp.concatenate([
        jnp.zeros(N_USERS * TOK, dtype=jnp.int32),
        jnp.ones(N_ITEMS * TOK, dtype=jnp.int32),
    ])
    col_parts = ([jnp.full((N_USERS,), c, dtype=jnp.int32) for c in range(TOK)]
                 + [jnp.full((N_ITEMS,), TOK + c, dtype=jnp.int32) for c in range(TOK)])
    col_idxs = jnp.concatenate(col_parts)
    return (x, node_idxs, col_idxs, table_idxs, N_USERS + N_ITEMS)
